# Initial kernel scaffold; baseline (speedup 1.0000x reference)
import jax
import jax.numpy as jnp
from jax.experimental import pallas as pl


def kernel(x, edge_index, edge_attr, W0, b0, W1, b1, W2, b2, Wc, bc):
    raise NotImplementedError("write your pallas kernel here")


# speedup vs baseline: 6.1061x; 6.1061x over previous
import functools

import jax
import jax.numpy as jnp
from jax import lax
from jax.experimental import pallas as pl
from jax.experimental.pallas import tpu as pltpu
from jax.experimental.pallas import tpu_sc as plsc

N = 10000
E = 160000
DIM = 256
HALF = 128

NC = 2
NS = 16
CHUNK = 128
CH = 80
E_PAD = NS * CH * CHUNK
TRASH = N

NPAD = 10240
ROWS_PER_TILE = NPAD // NS
SP_ROWS = NPAD

DEG_ROWS = NPAD
DEG_RPT = DEG_ROWS // NS
DEG_W = 128
DCH = CH // NC


def _mesh():
    return plsc.VectorSubcoreMesh(
        core_axis_name="c", subcore_axis_name="s",
        num_cores=NC, num_subcores=NS)


def _deg_body(cidx_hbm, zeros_hbm, ones_hbm, deg_hbm, cidx_v, ones_v, spd):
    c = lax.axis_index("c")
    s = lax.axis_index("s")
    pltpu.sync_copy(cidx_hbm.at[c, s], cidx_v)
    pltpu.sync_copy(ones_hbm, ones_v)
    base = s * DEG_RPT
    pltpu.sync_copy(zeros_hbm.at[pl.ds(base, DEG_RPT)], spd.at[pl.ds(base, DEG_RPT)])
    plsc.subcore_barrier()

    def chunk(k, carry):
        pltpu.sync_copy(ones_v, spd.at[cidx_v.at[k]], add=True)
        return carry

    lax.fori_loop(0, DCH, chunk, 0)
    plsc.subcore_barrier()
    pltpu.sync_copy(spd.at[pl.ds(base, DEG_RPT)], deg_hbm.at[c, pl.ds(base, DEG_RPT)])


def _prop_body(xp_hbm, ridx_hbm, cidx_hbm, out_hbm, ridx_v, cidx_v, buf, sp):
    c = lax.axis_index("c")
    s = lax.axis_index("s")
    pltpu.sync_copy(ridx_hbm.at[c, s], ridx_v)
    pltpu.sync_copy(cidx_hbm.at[s], cidx_v)
    base = s * ROWS_PER_TILE
    pltpu.sync_copy(xp_hbm.at[pl.ds(c * NPAD + base, ROWS_PER_TILE)],
                    sp.at[pl.ds(base, ROWS_PER_TILE)])
    plsc.subcore_barrier()

    def chunk(k, carry):
        pltpu.sync_copy(xp_hbm.at[ridx_v.at[k]], buf)
        pltpu.sync_copy(buf, sp.at[cidx_v.at[k]], add=True)
        return carry

    lax.fori_loop(0, CH, chunk, 0)
    plsc.subcore_barrier()
    pltpu.sync_copy(sp.at[pl.ds(base, ROWS_PER_TILE)],
                    out_hbm.at[c, pl.ds(base, ROWS_PER_TILE)])


def _make_deg():
    return functools.partial(
        pl.kernel,
        out_type=jax.ShapeDtypeStruct((NC, DEG_ROWS, DEG_W), jnp.float32),
        mesh=_mesh(),
        scratch_types=[
            pltpu.VMEM((DCH, CHUNK), jnp.int32),
            pltpu.VMEM((CHUNK, DEG_W), jnp.float32),
            pltpu.VMEM_SHARED((DEG_ROWS, DEG_W), jnp.float32),
        ],
    )(_deg_body)


def _deg_cidx(cidx):
    return cidx.reshape(NS, NC, DCH, CHUNK).transpose(1, 0, 2, 3)


def _make_prop():
    return functools.partial(
        pl.kernel,
        out_type=jax.ShapeDtypeStruct((NC, NPAD, HALF), jnp.float32),
        mesh=_mesh(),
        scratch_types=[
            pltpu.VMEM((CH, CHUNK), jnp.int32),
            pltpu.VMEM((CH, CHUNK), jnp.int32),
            pltpu.VMEM((CHUNK, HALF), jnp.float32),
            pltpu.VMEM_SHARED((SP_ROWS, HALF), jnp.float32),
        ],
    )(_prop_body)


BN = 1000


def _dense_body(x_ref, h1_ref, h2_ref, w0, b0r, w1, b1r, w2, b2r,
                c0, c1, c2, bcr, o_ref):
    acc = jnp.maximum(x_ref[...] @ w0[...] + b0r[...], 0.0) @ c0[...]
    acc = acc + jnp.maximum(h1_ref[...] @ w1[...] + b1r[...], 0.0) @ c1[...]
    acc = acc + jnp.maximum(h2_ref[...] @ w2[...] + b2r[...], 0.0) @ c2[...]
    o_ref[...] = acc + bcr[...]


def _dense(x, h1, h2, w0t, b0r, w1t, b1r, w2t, b2r, c0, c1, c2, bcr):
    mspec = pl.BlockSpec((BN, DIM), lambda i: (i, 0))
    wspec = pl.BlockSpec((DIM, DIM), lambda i: (0, 0))
    bspec = pl.BlockSpec((1, DIM), lambda i: (0, 0))
    return pl.pallas_call(
        _dense_body,
        grid=(N // BN,),
        in_specs=[mspec, mspec, mspec,
                  wspec, bspec, wspec, bspec, wspec, bspec,
                  wspec, wspec, wspec, bspec],
        out_specs=mspec,
        out_shape=jax.ShapeDtypeStruct((N, DIM), jnp.float32),
    )(x, h1, h2, w0t, b0r, w1t, b1r, w2t, b2r, c0, c1, c2, bcr)


def kernel(x, edge_index, edge_attr, W0, b0, W1, b1, W2, b2, Wc, bc):
    row = edge_index[0]
    col = edge_index[1]

    pad = E_PAD - E
    row_p = jnp.concatenate([row, jnp.zeros((pad,), row.dtype)])
    col_p = jnp.concatenate([col, jnp.full((pad,), TRASH, col.dtype)])
    cidx = col_p.reshape(NS, CH, CHUNK)
    r3 = row_p.reshape(NS, CH, CHUNK)
    ridx = jnp.stack([r3, r3 + NPAD])

    deg_kernel = _make_deg()
    prop_kernel = _make_prop()

    zeros2d = jnp.zeros((DEG_ROWS, DEG_W), jnp.float32)
    ones2d = jnp.ones((CHUNK, DEG_W), jnp.float32)
    dcounts = deg_kernel(_deg_cidx(cidx), zeros2d, ones2d)
    deg = dcounts[0, :N, 0] + dcounts[1, :N, 0] + 1.0
    dinv = lax.rsqrt(deg)

    zrows = jnp.zeros((NPAD - N, HALF), jnp.float32)
    g0 = dinv[:, None] * x
    g0_st = jnp.concatenate(
        [g0[:, :HALF], zrows, g0[:, HALF:], zrows], axis=0)
    s1 = prop_kernel(g0_st, ridx, cidx)
    h1 = dinv[:, None] * jnp.concatenate([s1[0, :N], s1[1, :N]], axis=1)

    d2 = (dinv * dinv)[:, None]
    g1_st = jnp.concatenate(
        [d2 * s1[0, :N], zrows, d2 * s1[1, :N], zrows], axis=0)
    s2 = prop_kernel(g1_st, ridx, cidx)
    h2 = dinv[:, None] * jnp.concatenate([s2[0, :N], s2[1, :N]], axis=1)

    out = _dense(x, h1, h2,
                 W0.T, b0[None], W1.T, b1[None], W2.T, b2[None],
                 Wc[:, :DIM].T, Wc[:, DIM:2 * DIM].T, Wc[:, 2 * DIM:].T,
                 bc[None])
    return (out, edge_attr)


# speedup vs baseline: 7.0982x; 1.1625x over previous
import functools

import jax
import jax.numpy as jnp
from jax import lax
from jax.experimental import pallas as pl
from jax.experimental.pallas import tpu as pltpu
from jax.experimental.pallas import tpu_sc as plsc

N = 10000
E = 160000
DIM = 256
HALF = 128

NC = 2
NS = 16
CHUNK = 128
CH = 80
CH2 = CH // 2
E_PAD = NS * CH * CHUNK
TRASH = N

NPAD = 10240
ROWS_PER_TILE = NPAD // NS
SP_ROWS = NPAD

DEG_ROWS = NPAD
DEG_RPT = DEG_ROWS // NS
DEG_W = 128
DCH = CH // NC


def _mesh():
    return plsc.VectorSubcoreMesh(
        core_axis_name="c", subcore_axis_name="s",
        num_cores=NC, num_subcores=NS)


def _deg_body(cidx_hbm, zeros_hbm, ones_hbm, deg_hbm, cidx_v, ones_v, spd):
    c = lax.axis_index("c")
    s = lax.axis_index("s")
    pltpu.sync_copy(cidx_hbm.at[c, s], cidx_v)
    pltpu.sync_copy(ones_hbm, ones_v)
    base = s * DEG_RPT
    pltpu.sync_copy(zeros_hbm.at[pl.ds(base, DEG_RPT)], spd.at[pl.ds(base, DEG_RPT)])
    plsc.subcore_barrier()

    def chunk(k, carry):
        pltpu.sync_copy(ones_v, spd.at[cidx_v.at[k]], add=True)
        return carry

    lax.fori_loop(0, DCH, chunk, 0)
    plsc.subcore_barrier()
    pltpu.sync_copy(spd.at[pl.ds(base, DEG_RPT)], deg_hbm.at[c, pl.ds(base, DEG_RPT)])


def _prop_body(xp_hbm, ridx_hbm, cidx_hbm, out_hbm, ridx_v, cidx_v,
               buf0, buf1, sp, gs0, gs1):
    c = lax.axis_index("c")
    s = lax.axis_index("s")
    pltpu.sync_copy(ridx_hbm.at[c, s], ridx_v)
    pltpu.sync_copy(cidx_hbm.at[s], cidx_v)
    base = s * ROWS_PER_TILE
    pltpu.sync_copy(xp_hbm.at[pl.ds(c * NPAD + base, ROWS_PER_TILE)],
                    sp.at[pl.ds(base, ROWS_PER_TILE)])
    plsc.subcore_barrier()

    bufs = (buf0, buf1)
    gsems = (gs0, gs1)
    for h in range(2):
        pltpu.sync_copy(ridx_hbm.at[c, s * 2 + h], ridx_v)
        off = h * CH2
        for b in range(2):
            pltpu.async_copy(xp_hbm.at[ridx_v.at[b]], bufs[b], gsems[b])

        def outer(i, carry, off=off):
            for b in range(2):
                k = i * 2 + b
                pltpu.make_async_copy(
                    xp_hbm.at[ridx_v.at[k]], bufs[b], gsems[b]).wait()
                pltpu.sync_copy(bufs[b], sp.at[cidx_v.at[off + k]], add=True)

                @pl.when(k + 2 < CH2)
                def _():
                    pltpu.async_copy(
                        xp_hbm.at[ridx_v.at[k + 2]], bufs[b], gsems[b])
            return carry

        lax.fori_loop(0, CH2 // 2, outer, 0)
    plsc.subcore_barrier()
    pltpu.sync_copy(sp.at[pl.ds(base, ROWS_PER_TILE)],
                    out_hbm.at[c, pl.ds(base, ROWS_PER_TILE)])


def _make_deg():
    return functools.partial(
        pl.kernel,
        out_type=jax.ShapeDtypeStruct((NC, DEG_ROWS, DEG_W), jnp.float32),
        mesh=_mesh(),
        scratch_types=[
            pltpu.VMEM((DCH, CHUNK), jnp.int32),
            pltpu.VMEM((CHUNK, DEG_W), jnp.float32),
            pltpu.VMEM_SHARED((DEG_ROWS, DEG_W), jnp.float32),
        ],
    )(_deg_body)


def _deg_cidx(cidx):
    return cidx.reshape(NS, NC, DCH, CHUNK).transpose(1, 0, 2, 3)


def _make_prop():
    return functools.partial(
        pl.kernel,
        out_type=jax.ShapeDtypeStruct((NC, NPAD, HALF), jnp.float32),
        mesh=_mesh(),
        scratch_types=[
            pltpu.VMEM((CH2, CHUNK), jnp.int32),
            pltpu.VMEM((CH, CHUNK), jnp.int32),
            pltpu.VMEM((CHUNK, HALF), jnp.float32),
            pltpu.VMEM((CHUNK, HALF), jnp.float32),
            pltpu.VMEM_SHARED((SP_ROWS, HALF), jnp.float32),
            pltpu.SemaphoreType.DMA,
            pltpu.SemaphoreType.DMA,
        ],
    )(_prop_body)


BN = 1000


def _dense_body(x_ref, h1_ref, h2_ref, w0, b0r, w1, b1r, w2, b2r,
                c0, c1, c2, bcr, o_ref):
    acc = jnp.maximum(x_ref[...] @ w0[...] + b0r[...], 0.0) @ c0[...]
    acc = acc + jnp.maximum(h1_ref[...] @ w1[...] + b1r[...], 0.0) @ c1[...]
    acc = acc + jnp.maximum(h2_ref[...] @ w2[...] + b2r[...], 0.0) @ c2[...]
    o_ref[...] = acc + bcr[...]


def _dense(x, h1, h2, w0t, b0r, w1t, b1r, w2t, b2r, c0, c1, c2, bcr):
    mspec = pl.BlockSpec((BN, DIM), lambda i: (i, 0))
    wspec = pl.BlockSpec((DIM, DIM), lambda i: (0, 0))
    bspec = pl.BlockSpec((1, DIM), lambda i: (0, 0))
    return pl.pallas_call(
        _dense_body,
        grid=(N // BN,),
        in_specs=[mspec, mspec, mspec,
                  wspec, bspec, wspec, bspec, wspec, bspec,
                  wspec, wspec, wspec, bspec],
        out_specs=mspec,
        out_shape=jax.ShapeDtypeStruct((N, DIM), jnp.float32),
    )(x, h1, h2, w0t, b0r, w1t, b1r, w2t, b2r, c0, c1, c2, bcr)


def kernel(x, edge_index, edge_attr, W0, b0, W1, b1, W2, b2, Wc, bc):
    row = edge_index[0]
    col = edge_index[1]

    pad = E_PAD - E
    row_p = jnp.concatenate([row, jnp.zeros((pad,), row.dtype)])
    col_p = jnp.concatenate([col, jnp.full((pad,), TRASH, col.dtype)])
    cidx = col_p.reshape(NS, CH, CHUNK)
    r3 = row_p.reshape(NS, CH, CHUNK)
    ridx = jnp.stack([r3, r3 + NPAD]).reshape(NC, NS * 2, CH2, CHUNK)

    deg_kernel = _make_deg()
    prop_kernel = _make_prop()

    zeros2d = jnp.zeros((DEG_ROWS, DEG_W), jnp.float32)
    ones2d = jnp.ones((CHUNK, DEG_W), jnp.float32)
    dcounts = deg_kernel(_deg_cidx(cidx), zeros2d, ones2d)
    deg = dcounts[0, :N, 0] + dcounts[1, :N, 0] + 1.0
    dinv = lax.rsqrt(deg)

    zrows = jnp.zeros((NPAD - N, HALF), jnp.float32)
    g0 = dinv[:, None] * x
    g0_st = jnp.concatenate(
        [g0[:, :HALF], zrows, g0[:, HALF:], zrows], axis=0)
    s1 = prop_kernel(g0_st, ridx, cidx)
    h1 = dinv[:, None] * jnp.concatenate([s1[0, :N], s1[1, :N]], axis=1)

    d2 = (dinv * dinv)[:, None]
    g1_st = jnp.concatenate(
        [d2 * s1[0, :N], zrows, d2 * s1[1, :N], zrows], axis=0)
    s2 = prop_kernel(g1_st, ridx, cidx)
    h2 = dinv[:, None] * jnp.concatenate([s2[0, :N], s2[1, :N]], axis=1)

    out = _dense(x, h1, h2,
                 W0.T, b0[None], W1.T, b1[None], W2.T, b2[None],
                 Wc[:, :DIM].T, Wc[:, DIM:2 * DIM].T, Wc[:, 2 * DIM:].T,
                 bc[None])
    return (out, edge_attr)


# speedup vs baseline: 8.8587x; 1.2480x over previous
import functools

import jax
import jax.numpy as jnp
from jax import lax
from jax.experimental import pallas as pl
from jax.experimental.pallas import tpu as pltpu
from jax.experimental.pallas import tpu_sc as plsc

N = 10000
E = 160000
DIM = 256
HALF = 128

NC = 2
NS = 16
CHUNK = 128
CH = 80
CH2 = CH // 2

NPAD = 10240
ROWS_PER_TILE = NPAD // NS
SP_ROWS = NPAD

DEG_ROWS = NPAD
DEG_RPT = DEG_ROWS // NS
DEG_W = 128
DCH = CH // NC


def _mesh():
    return plsc.VectorSubcoreMesh(
        core_axis_name="c", subcore_axis_name="s",
        num_cores=NC, num_subcores=NS)


def _deg_body(cidx_hbm, zeros_hbm, ones_hbm, deg_hbm, cidx_v, ones_v, spd):
    c = lax.axis_index("c")
    s = lax.axis_index("s")
    pltpu.sync_copy(cidx_hbm.at[c, s], cidx_v)
    pltpu.sync_copy(ones_hbm, ones_v)
    base = s * DEG_RPT
    pltpu.sync_copy(zeros_hbm.at[pl.ds(base, DEG_RPT)], spd.at[pl.ds(base, DEG_RPT)])
    plsc.subcore_barrier()

    def chunk(k, carry):
        pltpu.sync_copy(ones_v, spd.at[cidx_v.at[k]], add=True)
        return carry

    lax.fori_loop(0, DCH, chunk, 0)
    plsc.subcore_barrier()
    pltpu.sync_copy(spd.at[pl.ds(base, DEG_RPT)], deg_hbm.at[c, pl.ds(base, DEG_RPT)])


def _prop_body(xp_hbm, ridx_hbm, cidx_hbm, out_hbm, ridx_v, cidx_v,
               buf0, buf1, sp, gs0, gs1):
    c = lax.axis_index("c")
    s = lax.axis_index("s")
    pltpu.sync_copy(ridx_hbm.at[c, s], ridx_v)
    pltpu.sync_copy(cidx_hbm.at[s], cidx_v)
    base = s * ROWS_PER_TILE
    pltpu.sync_copy(xp_hbm.at[pl.ds(c * NPAD + base, ROWS_PER_TILE)],
                    sp.at[pl.ds(base, ROWS_PER_TILE)])
    plsc.subcore_barrier()

    bufs = (buf0, buf1)
    gsems = (gs0, gs1)
    for h in range(2):
        pltpu.sync_copy(ridx_hbm.at[c, s * 2 + h], ridx_v)
        off = h * CH2
        for b in range(2):
            pltpu.async_copy(xp_hbm.at[ridx_v.at[b]], bufs[b], gsems[b])

        def outer(i, carry, off=off):
            for b in range(2):
                k = i * 2 + b
                pltpu.make_async_copy(
                    xp_hbm.at[ridx_v.at[k]], bufs[b], gsems[b]).wait()
                pltpu.sync_copy(bufs[b], sp.at[cidx_v.at[off + k]], add=True)

                @pl.when(k + 2 < CH2)
                def _():
                    pltpu.async_copy(
                        xp_hbm.at[ridx_v.at[k + 2]], bufs[b], gsems[b])
            return carry

        lax.fori_loop(0, CH2 // 2, outer, 0)
    plsc.subcore_barrier()
    pltpu.sync_copy(sp.at[pl.ds(base, ROWS_PER_TILE)],
                    out_hbm.at[c, pl.ds(base, ROWS_PER_TILE)])


def _make_deg():
    return functools.partial(
        pl.kernel,
        out_type=jax.ShapeDtypeStruct((NC, DEG_ROWS, DEG_W), jnp.float32),
        mesh=_mesh(),
        scratch_types=[
            pltpu.VMEM((DCH, CHUNK), jnp.int32),
            pltpu.VMEM((CHUNK, DEG_W), jnp.float32),
            pltpu.VMEM_SHARED((DEG_ROWS, DEG_W), jnp.float32),
        ],
    )(_deg_body)


def _deg_cidx(cidx):
    return cidx.reshape(NS, NC, DCH, CHUNK).transpose(1, 0, 2, 3)


def _make_prop():
    return functools.partial(
        pl.kernel,
        out_type=jax.ShapeDtypeStruct((NC, NPAD, HALF), jnp.float32),
        mesh=_mesh(),
        scratch_types=[
            pltpu.VMEM((CH2, CHUNK), jnp.int32),
            pltpu.VMEM((CH, CHUNK), jnp.int32),
            pltpu.VMEM((CHUNK, HALF), jnp.float32),
            pltpu.VMEM((CHUNK, HALF), jnp.float32),
            pltpu.VMEM_SHARED((SP_ROWS, HALF), jnp.float32),
            pltpu.SemaphoreType.DMA,
            pltpu.SemaphoreType.DMA,
        ],
    )(_prop_body)


BN = 1000


def _dense_body(x_ref, h1_ref, h2_ref, w0, b0r, w1, b1r, w2, b2r,
                c0, c1, c2, bcr, o_ref):
    acc = jnp.maximum(x_ref[...] @ w0[...] + b0r[...], 0.0) @ c0[...]
    acc = acc + jnp.maximum(h1_ref[...] @ w1[...] + b1r[...], 0.0) @ c1[...]
    acc = acc + jnp.maximum(h2_ref[...] @ w2[...] + b2r[...], 0.0) @ c2[...]
    o_ref[...] = acc + bcr[...]


def _dense(x, h1, h2, w0t, b0r, w1t, b1r, w2t, b2r, c0, c1, c2, bcr):
    mspec = pl.BlockSpec((BN, DIM), lambda i: (i, 0))
    wspec = pl.BlockSpec((DIM, DIM), lambda i: (0, 0))
    bspec = pl.BlockSpec((1, DIM), lambda i: (0, 0))
    return pl.pallas_call(
        _dense_body,
        grid=(N // BN,),
        in_specs=[mspec, mspec, mspec,
                  wspec, bspec, wspec, bspec, wspec, bspec,
                  wspec, wspec, wspec, bspec],
        out_specs=mspec,
        out_shape=jax.ShapeDtypeStruct((N, DIM), jnp.float32),
    )(x, h1, h2, w0t, b0r, w1t, b1r, w2t, b2r, c0, c1, c2, bcr)


def kernel(x, edge_index, edge_attr, W0, b0, W1, b1, W2, b2, Wc, bc):
    row = edge_index[0]
    col = edge_index[1]

    EPT = E // NS
    PPT = CH * CHUNK - EPT
    row_t = row.reshape(NS, EPT)
    col_t = col.reshape(NS, EPT)
    padr = jnp.zeros((NS, PPT), row.dtype)
    padc = jnp.broadcast_to(jnp.arange(PPT, dtype=col.dtype) % (NPAD - N) + N,
                            (NS, PPT))
    row_p = jnp.concatenate([row_t, padr], axis=1)
    col_p = jnp.concatenate([col_t, padc], axis=1)
    cidx = col_p.reshape(NS, CH, CHUNK)
    r3 = row_p.reshape(NS, CH, CHUNK)
    ridx = jnp.stack([r3, r3 + NPAD]).reshape(NC, NS * 2, CH2, CHUNK)

    deg_kernel = _make_deg()
    prop_kernel = _make_prop()

    zeros2d = jnp.zeros((DEG_ROWS, DEG_W), jnp.float32)
    ones2d = jnp.ones((CHUNK, DEG_W), jnp.float32)
    dcounts = deg_kernel(_deg_cidx(cidx), zeros2d, ones2d)
    deg = dcounts[0, :N, 0] + dcounts[1, :N, 0] + 1.0
    dinv = lax.rsqrt(deg)

    zrows = jnp.zeros((NPAD - N, HALF), jnp.float32)
    g0 = dinv[:, None] * x
    g0_st = jnp.concatenate(
        [g0[:, :HALF], zrows, g0[:, HALF:], zrows], axis=0)
    s1 = prop_kernel(g0_st, ridx, cidx)
    h1 = dinv[:, None] * jnp.concatenate([s1[0, :N], s1[1, :N]], axis=1)

    d2 = (dinv * dinv)[:, None]
    g1_st = jnp.concatenate(
        [d2 * s1[0, :N], zrows, d2 * s1[1, :N], zrows], axis=0)
    s2 = prop_kernel(g1_st, ridx, cidx)
    h2 = dinv[:, None] * jnp.concatenate([s2[0, :N], s2[1, :N]], axis=1)

    out = _dense(x, h1, h2,
                 W0.T, b0[None], W1.T, b1[None], W2.T, b2[None],
                 Wc[:, :DIM].T, Wc[:, DIM:2 * DIM].T, Wc[:, 2 * DIM:].T,
                 bc[None])
    return (out, edge_attr)


# speedup vs baseline: 8.8806x; 1.0025x over previous
import functools

import jax
import jax.numpy as jnp
from jax import lax
from jax.experimental import pallas as pl
from jax.experimental.pallas import tpu as pltpu
from jax.experimental.pallas import tpu_sc as plsc

N = 10000
E = 160000
DIM = 256
HALF = 128

NC = 2
NS = 16
CHUNK = 128
CH = 80
CH2 = CH // 2
SUB = CHUNK // 2

NPAD = 10240
ROWS_PER_TILE = NPAD // NS
SP_ROWS = NPAD

DEG_ROWS = NPAD
DEG_RPT = DEG_ROWS // NS
DEG_W = 128
DCH = CH // NC


def _mesh():
    return plsc.VectorSubcoreMesh(
        core_axis_name="c", subcore_axis_name="s",
        num_cores=NC, num_subcores=NS)


def _deg_body(cidx_hbm, zeros_hbm, ones_hbm, deg_hbm, cidx_v, ones_v, spd):
    c = lax.axis_index("c")
    s = lax.axis_index("s")
    pltpu.sync_copy(cidx_hbm.at[c, s], cidx_v)
    pltpu.sync_copy(ones_hbm, ones_v)
    base = s * DEG_RPT
    pltpu.sync_copy(zeros_hbm.at[pl.ds(base, DEG_RPT)], spd.at[pl.ds(base, DEG_RPT)])
    plsc.subcore_barrier()

    def chunk(k, carry):
        pltpu.sync_copy(ones_v, spd.at[cidx_v.at[k]], add=True)
        return carry

    lax.fori_loop(0, DCH, chunk, 0)
    plsc.subcore_barrier()
    pltpu.sync_copy(spd.at[pl.ds(base, DEG_RPT)], deg_hbm.at[c, pl.ds(base, DEG_RPT)])


def _prop_body(xp_hbm, ridx_hbm, cidx_hbm, out_hbm, ridx_v, cidx_v,
               buf0, buf1, sp, gs0, gs1):
    c = lax.axis_index("c")
    s = lax.axis_index("s")
    pltpu.sync_copy(cidx_hbm.at[s], cidx_v)
    base = s * ROWS_PER_TILE
    pltpu.sync_copy(xp_hbm.at[pl.ds(c * NPAD + base, ROWS_PER_TILE)],
                    sp.at[pl.ds(base, ROWS_PER_TILE)])
    plsc.subcore_barrier()

    bufs = (buf0, buf1)
    gsems = (gs0, gs1)
    for h in range(2):
        pltpu.sync_copy(ridx_hbm.at[c, s * 2 + h], ridx_v)
        off = h * CH2
        for b in range(2):
            for j in range(2):
                pltpu.async_copy(xp_hbm.at[ridx_v.at[b, pl.ds(j * SUB, SUB)]],
                                 bufs[b].at[pl.ds(j * SUB, SUB)], gsems[b])

        def outer(i, carry, off=off):
            for b in range(2):
                k = i * 2 + b
                for j in range(2):
                    pltpu.make_async_copy(
                        xp_hbm.at[ridx_v.at[k, pl.ds(j * SUB, SUB)]],
                        bufs[b].at[pl.ds(j * SUB, SUB)], gsems[b]).wait()
                pltpu.sync_copy(bufs[b], sp.at[cidx_v.at[off + k]], add=True)

                @pl.when(k + 2 < CH2)
                def _():
                    for j in range(2):
                        pltpu.async_copy(
                            xp_hbm.at[ridx_v.at[k + 2, pl.ds(j * SUB, SUB)]],
                            bufs[b].at[pl.ds(j * SUB, SUB)], gsems[b])
            return carry

        lax.fori_loop(0, CH2 // 2, outer, 0)
    plsc.subcore_barrier()
    pltpu.sync_copy(sp.at[pl.ds(base, ROWS_PER_TILE)],
                    out_hbm.at[c, pl.ds(base, ROWS_PER_TILE)])


def _make_deg():
    return functools.partial(
        pl.kernel,
        out_type=jax.ShapeDtypeStruct((NC, DEG_ROWS, DEG_W), jnp.float32),
        mesh=_mesh(),
        scratch_types=[
            pltpu.VMEM((DCH, CHUNK), jnp.int32),
            pltpu.VMEM((CHUNK, DEG_W), jnp.float32),
            pltpu.VMEM_SHARED((DEG_ROWS, DEG_W), jnp.float32),
        ],
    )(_deg_body)


def _deg_cidx(cidx):
    return cidx.reshape(NS, NC, DCH, CHUNK).transpose(1, 0, 2, 3)


def _make_prop():
    return functools.partial(
        pl.kernel,
        out_type=jax.ShapeDtypeStruct((NC, NPAD, HALF), jnp.float32),
        mesh=_mesh(),
        scratch_types=[
            pltpu.VMEM((CH2, CHUNK), jnp.int32),
            pltpu.VMEM((CH, CHUNK), jnp.int32),
            pltpu.VMEM((CHUNK, HALF), jnp.float32),
            pltpu.VMEM((CHUNK, HALF), jnp.float32),
            pltpu.VMEM_SHARED((SP_ROWS, HALF), jnp.float32),
            pltpu.SemaphoreType.DMA,
            pltpu.SemaphoreType.DMA,
        ],
    )(_prop_body)


BN = 1000


def _dense_body(x_ref, h1_ref, h2_ref, w0, b0r, w1, b1r, w2, b2r,
                c0, c1, c2, bcr, o_ref):
    acc = jnp.maximum(x_ref[...] @ w0[...] + b0r[...], 0.0) @ c0[...]
    acc = acc + jnp.maximum(h1_ref[...] @ w1[...] + b1r[...], 0.0) @ c1[...]
    acc = acc + jnp.maximum(h2_ref[...] @ w2[...] + b2r[...], 0.0) @ c2[...]
    o_ref[...] = acc + bcr[...]


def _dense(x, h1, h2, w0t, b0r, w1t, b1r, w2t, b2r, c0, c1, c2, bcr):
    mspec = pl.BlockSpec((BN, DIM), lambda i: (i, 0))
    wspec = pl.BlockSpec((DIM, DIM), lambda i: (0, 0))
    bspec = pl.BlockSpec((1, DIM), lambda i: (0, 0))
    return pl.pallas_call(
        _dense_body,
        grid=(N // BN,),
        in_specs=[mspec, mspec, mspec,
                  wspec, bspec, wspec, bspec, wspec, bspec,
                  wspec, wspec, wspec, bspec],
        out_specs=mspec,
        out_shape=jax.ShapeDtypeStruct((N, DIM), jnp.float32),
    )(x, h1, h2, w0t, b0r, w1t, b1r, w2t, b2r, c0, c1, c2, bcr)


def kernel(x, edge_index, edge_attr, W0, b0, W1, b1, W2, b2, Wc, bc):
    row = edge_index[0]
    col = edge_index[1]

    EPT = E // NS
    PPT = CH * CHUNK - EPT
    row_t = row.reshape(NS, EPT)
    col_t = col.reshape(NS, EPT)
    padr = jnp.zeros((NS, PPT), row.dtype)
    padc = jnp.broadcast_to(jnp.arange(PPT, dtype=col.dtype) % (NPAD - N) + N,
                            (NS, PPT))
    row_p = jnp.concatenate([row_t, padr], axis=1)
    col_p = jnp.concatenate([col_t, padc], axis=1)
    cidx = col_p.reshape(NS, CH, CHUNK)
    r3 = row_p.reshape(NS, CH, CHUNK)
    ridx = jnp.stack([r3, r3 + NPAD]).reshape(NC, NS * 2, CH2, CHUNK)

    deg_kernel = _make_deg()
    prop_kernel = _make_prop()

    zeros2d = jnp.zeros((DEG_ROWS, DEG_W), jnp.float32)
    ones2d = jnp.ones((CHUNK, DEG_W), jnp.float32)
    dcounts = deg_kernel(_deg_cidx(cidx), zeros2d, ones2d)
    deg = dcounts[0, :N, 0] + dcounts[1, :N, 0] + 1.0
    dinv = lax.rsqrt(deg)

    zrows = jnp.zeros((NPAD - N, HALF), jnp.float32)
    g0 = dinv[:, None] * x
    g0_st = jnp.concatenate(
        [g0[:, :HALF], zrows, g0[:, HALF:], zrows], axis=0)
    s1 = prop_kernel(g0_st, ridx, cidx)
    h1 = dinv[:, None] * jnp.concatenate([s1[0, :N], s1[1, :N]], axis=1)

    d2 = (dinv * dinv)[:, None]
    g1_st = jnp.concatenate(
        [d2 * s1[0, :N], zrows, d2 * s1[1, :N], zrows], axis=0)
    s2 = prop_kernel(g1_st, ridx, cidx)
    h2 = dinv[:, None] * jnp.concatenate([s2[0, :N], s2[1, :N]], axis=1)

    out = _dense(x, h1, h2,
                 W0.T, b0[None], W1.T, b1[None], W2.T, b2[None],
                 Wc[:, :DIM].T, Wc[:, DIM:2 * DIM].T, Wc[:, 2 * DIM:].T,
                 bc[None])
    return (out, edge_attr)


# speedup vs baseline: 9.2964x; 1.0468x over previous
import functools

import jax
import jax.numpy as jnp
from jax import lax
from jax.experimental import pallas as pl
from jax.experimental.pallas import tpu as pltpu
from jax.experimental.pallas import tpu_sc as plsc

N = 10000
E = 160000
DIM = 256
HALF = 128

NC = 2
NS = 16
CHUNK = 128
CH = 80
CH2 = CH // 2
SUB = CHUNK // 2

NPAD = 10240
ROWS_PER_TILE = NPAD // NS
SP_ROWS = NPAD

DEG_ROWS = NPAD
DEG_RPT = DEG_ROWS // NS
DEG_W = 128
DCH = CH // NC


def _mesh():
    return plsc.VectorSubcoreMesh(
        core_axis_name="c", subcore_axis_name="s",
        num_cores=NC, num_subcores=NS)


def _deg_body(cidx_hbm, zeros_hbm, ones_hbm, deg_hbm, cidx_v, ones_v, spd):
    c = lax.axis_index("c")
    s = lax.axis_index("s")
    pltpu.sync_copy(cidx_hbm.at[c, s], cidx_v)
    pltpu.sync_copy(ones_hbm, ones_v)
    base = s * DEG_RPT
    pltpu.sync_copy(zeros_hbm.at[pl.ds(base, DEG_RPT)], spd.at[pl.ds(base, DEG_RPT)])
    plsc.subcore_barrier()

    def chunk(k, carry):
        pltpu.sync_copy(ones_v, spd.at[cidx_v.at[k]], add=True)
        return carry

    lax.fori_loop(0, DCH, chunk, 0)
    plsc.subcore_barrier()
    pltpu.sync_copy(spd.at[pl.ds(base, DEG_RPT)], deg_hbm.at[c, pl.ds(base, DEG_RPT)])


def _prop_body(xp_hbm, ridx_hbm, cidx_hbm, out_hbm, ridx_v, cidx_v,
               buf0, buf1, sp, gs0, gs1):
    c = lax.axis_index("c")
    s = lax.axis_index("s")
    pltpu.sync_copy(cidx_hbm.at[s], cidx_v)
    base = s * ROWS_PER_TILE
    pltpu.sync_copy(xp_hbm.at[pl.ds(c * NPAD + base, ROWS_PER_TILE)],
                    sp.at[pl.ds(base, ROWS_PER_TILE)])
    plsc.subcore_barrier()

    bufs = (buf0, buf1)
    gsems = (gs0, gs1)
    for h in range(2):
        pltpu.sync_copy(ridx_hbm.at[c, s * 2 + h], ridx_v)
        off = h * CH2
        for b in range(2):
            for j in range(2):
                pltpu.async_copy(xp_hbm.at[ridx_v.at[b, pl.ds(j * SUB, SUB)]],
                                 bufs[b].at[pl.ds(j * SUB, SUB)], gsems[b])

        def outer(i, carry, off=off):
            for b in range(2):
                k = i * 2 + b
                for j in range(2):
                    pltpu.make_async_copy(
                        xp_hbm.at[ridx_v.at[k, pl.ds(j * SUB, SUB)]],
                        bufs[b].at[pl.ds(j * SUB, SUB)], gsems[b]).wait()
                pltpu.sync_copy(bufs[b], sp.at[cidx_v.at[off + k]], add=True)

                @pl.when(k + 2 < CH2)
                def _():
                    for j in range(2):
                        pltpu.async_copy(
                            xp_hbm.at[ridx_v.at[k + 2, pl.ds(j * SUB, SUB)]],
                            bufs[b].at[pl.ds(j * SUB, SUB)], gsems[b])
            return carry

        lax.fori_loop(0, CH2 // 2, outer, 0)
    plsc.subcore_barrier()
    pltpu.sync_copy(sp.at[pl.ds(base, ROWS_PER_TILE)],
                    out_hbm.at[c, pl.ds(base, ROWS_PER_TILE)])


def _make_deg():
    return functools.partial(
        pl.kernel,
        out_type=jax.ShapeDtypeStruct((NC, DEG_ROWS, DEG_W), jnp.float32),
        mesh=_mesh(),
        scratch_types=[
            pltpu.VMEM((DCH, CHUNK), jnp.int32),
            pltpu.VMEM((CHUNK, DEG_W), jnp.float32),
            pltpu.VMEM_SHARED((DEG_ROWS, DEG_W), jnp.float32),
        ],
    )(_deg_body)


def _deg_cidx(cidx):
    return cidx.reshape(NS, NC, DCH, CHUNK).transpose(1, 0, 2, 3)


def _make_prop():
    return functools.partial(
        pl.kernel,
        out_type=jax.ShapeDtypeStruct((NC, NPAD, HALF), jnp.float32),
        mesh=_mesh(),
        scratch_types=[
            pltpu.VMEM((CH2, CHUNK), jnp.int32),
            pltpu.VMEM((CH, CHUNK), jnp.int32),
            pltpu.VMEM((CHUNK, HALF), jnp.float32),
            pltpu.VMEM((CHUNK, HALF), jnp.float32),
            pltpu.VMEM_SHARED((SP_ROWS, HALF), jnp.float32),
            pltpu.SemaphoreType.DMA,
            pltpu.SemaphoreType.DMA,
        ],
    )(_prop_body)


BN = 1000


_mspec = pl.BlockSpec((BN, DIM), lambda i: (i, 0))
_sspec = lambda c: pl.BlockSpec((1, BN, HALF), lambda i: (c, i, 0))
_wspec = pl.BlockSpec((DIM, DIM), lambda i: (0, 0))
_hspec = pl.BlockSpec((HALF, DIM), lambda i: (0, 0))
_bspec = pl.BlockSpec((1, DIM), lambda i: (0, 0))
_dspec = pl.BlockSpec((BN, 1), lambda i: (i, 0))


def _dense1_body(x_ref, sa, sb, dv, w0, b0r, w1a, w1b, b1r, c0, c1, bcr,
                 o_ref):
    acc = jnp.maximum(x_ref[...] @ w0[...] + b0r[...], 0.0) @ c0[...]
    ha = dv[...] * sa[0]
    hb = dv[...] * sb[0]
    acc = acc + jnp.maximum(ha @ w1a[...] + hb @ w1b[...] + b1r[...],
                            0.0) @ c1[...]
    o_ref[...] = acc + bcr[...]


def _dense2_body(acc_ref, sa, sb, dv, w2a, w2b, b2r, c2, o_ref):
    ha = dv[...] * sa[0]
    hb = dv[...] * sb[0]
    o_ref[...] = acc_ref[...] + jnp.maximum(
        ha @ w2a[...] + hb @ w2b[...] + b2r[...], 0.0) @ c2[...]


def _dense1(x, s1, dinv2, w0t, b0r, w1ta, w1tb, b1r, c0, c1, bcr):
    return pl.pallas_call(
        _dense1_body,
        grid=(N // BN,),
        in_specs=[_mspec, _sspec(0), _sspec(1), _dspec,
                  _wspec, _bspec, _hspec, _hspec, _bspec,
                  _wspec, _wspec, _bspec],
        out_specs=_mspec,
        out_shape=jax.ShapeDtypeStruct((N, DIM), jnp.float32),
    )(x, s1, s1, dinv2, w0t, b0r, w1ta, w1tb, b1r, c0, c1, bcr)


def _dense2(acc, s2, dinv2, w2ta, w2tb, b2r, c2):
    return pl.pallas_call(
        _dense2_body,
        grid=(N // BN,),
        in_specs=[_mspec, _sspec(0), _sspec(1), _dspec,
                  _hspec, _hspec, _bspec, _wspec],
        out_specs=_mspec,
        out_shape=jax.ShapeDtypeStruct((N, DIM), jnp.float32),
    )(acc, s2, s2, dinv2, w2ta, w2tb, b2r, c2)


def kernel(x, edge_index, edge_attr, W0, b0, W1, b1, W2, b2, Wc, bc):
    row = edge_index[0]
    col = edge_index[1]

    EPT = E // NS
    PPT = CH * CHUNK - EPT
    row_t = row.reshape(NS, EPT)
    col_t = col.reshape(NS, EPT)
    padr = jnp.zeros((NS, PPT), row.dtype)
    padc = jnp.broadcast_to(jnp.arange(PPT, dtype=col.dtype) % (NPAD - N) + N,
                            (NS, PPT))
    row_p = jnp.concatenate([row_t, padr], axis=1)
    col_p = jnp.concatenate([col_t, padc], axis=1)
    cidx = col_p.reshape(NS, CH, CHUNK)
    r3 = row_p.reshape(NS, CH, CHUNK)
    ridx = jnp.stack([r3, r3 + NPAD]).reshape(NC, NS * 2, CH2, CHUNK)

    deg_kernel = _make_deg()
    prop_kernel = _make_prop()

    zeros2d = jnp.zeros((DEG_ROWS, DEG_W), jnp.float32)
    ones2d = jnp.ones((CHUNK, DEG_W), jnp.float32)
    dcounts = deg_kernel(_deg_cidx(cidx), zeros2d, ones2d)
    deg = dcounts[0, :N, 0] + dcounts[1, :N, 0] + 1.0
    dinv = lax.rsqrt(deg)

    zrows = jnp.zeros((NPAD - N, HALF), jnp.float32)
    g0 = dinv[:, None] * x
    g0_st = jnp.concatenate(
        [g0[:, :HALF], zrows, g0[:, HALF:], zrows], axis=0)
    s1 = prop_kernel(g0_st, ridx, cidx)

    d2pad = jnp.pad(dinv * dinv, (0, NPAD - N))[None, :, None]
    g1_st = (d2pad * s1).reshape(NC * NPAD, HALF)
    s2 = prop_kernel(g1_st, ridx, cidx)

    dinv2 = dinv[:, None]
    acc = _dense1(x, s1, dinv2, W0.T, b0[None], W1.T[:HALF], W1.T[HALF:],
                  b1[None], Wc[:, :DIM].T, Wc[:, DIM:2 * DIM].T, bc[None])
    out = _dense2(acc, s2, dinv2, W2.T[:HALF], W2.T[HALF:], b2[None],
                  Wc[:, 2 * DIM:].T)
    return (out, edge_attr)


# speedup vs baseline: 9.3010x; 1.0005x over previous
import functools

import jax
import jax.numpy as jnp
from jax import lax
from jax.experimental import pallas as pl
from jax.experimental.pallas import tpu as pltpu
from jax.experimental.pallas import tpu_sc as plsc

N = 10000
E = 160000
DIM = 256
HALF = 128

NC = 2
NS = 16
CHUNK = 128
CH = 80
CH2 = CH // 2
NSUB = 4
SUB = CHUNK // NSUB

NPAD = 10240
ROWS_PER_TILE = NPAD // NS
SP_ROWS = NPAD

DEG_ROWS = NPAD
DEG_RPT = DEG_ROWS // NS
DEG_W = 128
DCH = CH // NC


def _mesh():
    return plsc.VectorSubcoreMesh(
        core_axis_name="c", subcore_axis_name="s",
        num_cores=NC, num_subcores=NS)


def _deg_body(cidx_hbm, zeros_hbm, ones_hbm, deg_hbm, cidx_v, ones_v, spd):
    c = lax.axis_index("c")
    s = lax.axis_index("s")
    pltpu.sync_copy(cidx_hbm.at[c, s], cidx_v)
    pltpu.sync_copy(ones_hbm, ones_v)
    base = s * DEG_RPT
    pltpu.sync_copy(zeros_hbm.at[pl.ds(base, DEG_RPT)], spd.at[pl.ds(base, DEG_RPT)])
    plsc.subcore_barrier()

    def chunk(k, carry):
        pltpu.sync_copy(ones_v, spd.at[cidx_v.at[k]], add=True)
        return carry

    lax.fori_loop(0, DCH, chunk, 0)
    plsc.subcore_barrier()
    pltpu.sync_copy(spd.at[pl.ds(base, DEG_RPT)], deg_hbm.at[c, pl.ds(base, DEG_RPT)])


def _prop_body(xp_hbm, ridx_hbm, cidx_hbm, out_hbm, ridx_v, cidx_v,
               buf0, buf1, sp, gs0, gs1):
    c = lax.axis_index("c")
    s = lax.axis_index("s")
    pltpu.sync_copy(cidx_hbm.at[s], cidx_v)
    base = s * ROWS_PER_TILE
    pltpu.sync_copy(xp_hbm.at[pl.ds(c * NPAD + base, ROWS_PER_TILE)],
                    sp.at[pl.ds(base, ROWS_PER_TILE)])
    plsc.subcore_barrier()

    bufs = (buf0, buf1)
    gsems = (gs0, gs1)
    for h in range(2):
        pltpu.sync_copy(ridx_hbm.at[c, s * 2 + h], ridx_v)
        off = h * CH2
        for b in range(2):
            for j in range(NSUB):
                pltpu.async_copy(xp_hbm.at[ridx_v.at[b, pl.ds(j * SUB, SUB)]],
                                 bufs[b].at[pl.ds(j * SUB, SUB)], gsems[b])

        def outer(i, carry, off=off):
            for b in range(2):
                k = i * 2 + b
                for j in range(NSUB):
                    pltpu.make_async_copy(
                        xp_hbm.at[ridx_v.at[k, pl.ds(j * SUB, SUB)]],
                        bufs[b].at[pl.ds(j * SUB, SUB)], gsems[b]).wait()
                pltpu.sync_copy(bufs[b], sp.at[cidx_v.at[off + k]], add=True)

                @pl.when(k + 2 < CH2)
                def _():
                    for j in range(NSUB):
                        pltpu.async_copy(
                            xp_hbm.at[ridx_v.at[k + 2, pl.ds(j * SUB, SUB)]],
                            bufs[b].at[pl.ds(j * SUB, SUB)], gsems[b])
            return carry

        lax.fori_loop(0, CH2 // 2, outer, 0)
    plsc.subcore_barrier()
    pltpu.sync_copy(sp.at[pl.ds(base, ROWS_PER_TILE)],
                    out_hbm.at[c, pl.ds(base, ROWS_PER_TILE)])


def _make_deg():
    return functools.partial(
        pl.kernel,
        out_type=jax.ShapeDtypeStruct((NC, DEG_ROWS, DEG_W), jnp.float32),
        mesh=_mesh(),
        scratch_types=[
            pltpu.VMEM((DCH, CHUNK), jnp.int32),
            pltpu.VMEM((CHUNK, DEG_W), jnp.float32),
            pltpu.VMEM_SHARED((DEG_ROWS, DEG_W), jnp.float32),
        ],
    )(_deg_body)


def _deg_cidx(cidx):
    return cidx.reshape(NS, NC, DCH, CHUNK).transpose(1, 0, 2, 3)


def _make_prop():
    return functools.partial(
        pl.kernel,
        out_type=jax.ShapeDtypeStruct((NC, NPAD, HALF), jnp.float32),
        mesh=_mesh(),
        scratch_types=[
            pltpu.VMEM((CH2, CHUNK), jnp.int32),
            pltpu.VMEM((CH, CHUNK), jnp.int32),
            pltpu.VMEM((CHUNK, HALF), jnp.float32),
            pltpu.VMEM((CHUNK, HALF), jnp.float32),
            pltpu.VMEM_SHARED((SP_ROWS, HALF), jnp.float32),
            pltpu.SemaphoreType.DMA,
            pltpu.SemaphoreType.DMA,
        ],
    )(_prop_body)


BN = 1000


_mspec = pl.BlockSpec((BN, DIM), lambda i: (i, 0))
_sspec = lambda c: pl.BlockSpec((1, BN, HALF), lambda i: (c, i, 0))
_wspec = pl.BlockSpec((DIM, DIM), lambda i: (0, 0))
_hspec = pl.BlockSpec((HALF, DIM), lambda i: (0, 0))
_bspec = pl.BlockSpec((1, DIM), lambda i: (0, 0))
_dspec = pl.BlockSpec((BN, 1), lambda i: (i, 0))


def _dense1_body(x_ref, sa, sb, dv, w0, b0r, w1a, w1b, b1r, c0, c1, bcr,
                 o_ref):
    acc = jnp.maximum(x_ref[...] @ w0[...] + b0r[...], 0.0) @ c0[...]
    ha = dv[...] * sa[0]
    hb = dv[...] * sb[0]
    acc = acc + jnp.maximum(ha @ w1a[...] + hb @ w1b[...] + b1r[...],
                            0.0) @ c1[...]
    o_ref[...] = acc + bcr[...]


def _dense2_body(acc_ref, sa, sb, dv, w2a, w2b, b2r, c2, o_ref):
    ha = dv[...] * sa[0]
    hb = dv[...] * sb[0]
    o_ref[...] = acc_ref[...] + jnp.maximum(
        ha @ w2a[...] + hb @ w2b[...] + b2r[...], 0.0) @ c2[...]


def _dense1(x, s1, dinv2, w0t, b0r, w1ta, w1tb, b1r, c0, c1, bcr):
    return pl.pallas_call(
        _dense1_body,
        grid=(N // BN,),
        in_specs=[_mspec, _sspec(0), _sspec(1), _dspec,
                  _wspec, _bspec, _hspec, _hspec, _bspec,
                  _wspec, _wspec, _bspec],
        out_specs=_mspec,
        out_shape=jax.ShapeDtypeStruct((N, DIM), jnp.float32),
    )(x, s1, s1, dinv2, w0t, b0r, w1ta, w1tb, b1r, c0, c1, bcr)


def _dense2(acc, s2, dinv2, w2ta, w2tb, b2r, c2):
    return pl.pallas_call(
        _dense2_body,
        grid=(N // BN,),
        in_specs=[_mspec, _sspec(0), _sspec(1), _dspec,
                  _hspec, _hspec, _bspec, _wspec],
        out_specs=_mspec,
        out_shape=jax.ShapeDtypeStruct((N, DIM), jnp.float32),
    )(acc, s2, s2, dinv2, w2ta, w2tb, b2r, c2)


def kernel(x, edge_index, edge_attr, W0, b0, W1, b1, W2, b2, Wc, bc):
    row = edge_index[0]
    col = edge_index[1]

    EPT = E // NS
    PPT = CH * CHUNK - EPT
    row_t = row.reshape(NS, EPT)
    col_t = col.reshape(NS, EPT)
    padr = jnp.zeros((NS, PPT), row.dtype)
    padc = jnp.broadcast_to(jnp.arange(PPT, dtype=col.dtype) % (NPAD - N) + N,
                            (NS, PPT))
    row_p = jnp.concatenate([row_t, padr], axis=1)
    col_p = jnp.concatenate([col_t, padc], axis=1)
    cidx = col_p.reshape(NS, CH, CHUNK)
    r3 = row_p.reshape(NS, CH, CHUNK)
    ridx = jnp.stack([r3, r3 + NPAD]).reshape(NC, NS * 2, CH2, CHUNK)

    deg_kernel = _make_deg()
    prop_kernel = _make_prop()

    zeros2d = jnp.zeros((DEG_ROWS, DEG_W), jnp.float32)
    ones2d = jnp.ones((CHUNK, DEG_W), jnp.float32)
    dcounts = deg_kernel(_deg_cidx(cidx), zeros2d, ones2d)
    deg = dcounts[0, :N, 0] + dcounts[1, :N, 0] + 1.0
    dinv = lax.rsqrt(deg)

    zrows = jnp.zeros((NPAD - N, HALF), jnp.float32)
    g0 = dinv[:, None] * x
    g0_st = jnp.concatenate(
        [g0[:, :HALF], zrows, g0[:, HALF:], zrows], axis=0)
    s1 = prop_kernel(g0_st, ridx, cidx)

    d2pad = jnp.pad(dinv * dinv, (0, NPAD - N))[None, :, None]
    g1_st = (d2pad * s1).reshape(NC * NPAD, HALF)
    s2 = prop_kernel(g1_st, ridx, cidx)

    dinv2 = dinv[:, None]
    acc = _dense1(x, s1, dinv2, W0.T, b0[None], W1.T[:HALF], W1.T[HALF:],
                  b1[None], Wc[:, :DIM].T, Wc[:, DIM:2 * DIM].T, bc[None])
    out = _dense2(acc, s2, dinv2, W2.T[:HALF], W2.T[HALF:], b2[None],
                  Wc[:, 2 * DIM:].T)
    return (out, edge_attr)
